# baseline (speedup 1.0000x reference)
import functools

import jax
import jax.numpy as jnp
from jax import lax
from jax.experimental import pallas as pl
from jax.experimental.pallas import tpu as pltpu
from jax.experimental.pallas import tpu_sc as plsc

E = 8
H = 1024
I_DIM = 512
IS_DIM = 512
T = 2048
TM = 128
NT = 40
P = NT * TM
NSUB = 32


def _k1_body(x_ref, gate_w_ref, swg_ref, swu_ref, swd_ref,
             shared_ref, xb_ref, slot0_ref, slot1_ref, w1_ref, w2_ref,
             te_ref):
    x32 = x_ref[...]
    xb = x32.astype(jnp.bfloat16)
    xb_ref[...] = xb

    logits = jax.lax.dot_general(
        x32, gate_w_ref[...], (((1,), (1,)), ((), ())),
        preferred_element_type=jnp.float32)
    m = jnp.max(logits, axis=-1, keepdims=True)
    p = jnp.exp(logits - m)
    p = p / jnp.sum(p, axis=-1, keepdims=True)
    lane = jax.lax.broadcasted_iota(jnp.int32, (T, E), 1)
    v1 = jnp.max(p, axis=-1, keepdims=True)
    e1 = jnp.min(jnp.where(p >= v1, lane, E), axis=-1, keepdims=True)
    m1 = lane == e1
    pm = jnp.where(m1, -jnp.inf, p)
    v2 = jnp.max(pm, axis=-1, keepdims=True)
    e2 = jnp.min(jnp.where(pm >= v2, lane, E), axis=-1, keepdims=True)
    m2 = lane == e2
    s = v1 + v2 + 1e-20
    w1_ref[...] = v1 / s
    w2_ref[...] = v2 / s

    oh1 = m1.astype(jnp.float32)
    oh2 = m2.astype(jnp.float32)

    r = jax.lax.broadcasted_iota(jnp.int32, (TM, TM), 0)
    c = jax.lax.broadcasted_iota(jnp.int32, (TM, TM), 1)
    tril = (r > c).astype(jnp.float32)
    def excl_cumsum(oh):
        out, off = [], jnp.zeros((1, E), jnp.float32)
        for b in range(T // TM):
            blk = oh[b * TM:(b + 1) * TM]
            out.append(jnp.dot(tril, blk, preferred_element_type=jnp.float32)
                       + off)
            off = off + jnp.sum(blk, axis=0, keepdims=True)
        return jnp.concatenate(out, axis=0), off
    C1, tot1 = excl_cumsum(oh1)
    C2, tot2 = excl_cumsum(oh2)

    counts = tot1 + tot2
    ntiles = jnp.floor((counts + (TM - 1)) * (1.0 / TM))
    fe = jax.lax.broadcasted_iota(jnp.int32, (E, E), 0)
    ee = jax.lax.broadcasted_iota(jnp.int32, (E, E), 1)
    lt = (fe < ee).astype(jnp.float32)
    tbase_tiles = jnp.dot(ntiles, lt,
                          preferred_element_type=jnp.float32)
    base = tbase_tiles * float(TM)

    rank0 = jnp.sum(C1 * oh1, axis=-1, keepdims=True)
    rank1 = jnp.sum((tot1 + C2) * oh2, axis=-1, keepdims=True)
    base0 = jnp.sum(base * oh1, axis=-1, keepdims=True)
    base1 = jnp.sum(base * oh2, axis=-1, keepdims=True)
    slot0_ref[...] = (base0 + rank0).astype(jnp.int32)
    slot1_ref[...] = (base1 + rank1).astype(jnp.int32)

    j64 = jax.lax.broadcasted_iota(jnp.int32, (8, 64), 1)
    tbi = tbase_tiles.astype(jnp.int32)
    te = jnp.zeros((8, 64), jnp.int32)
    for e in range(E):
        tb_e = tbi[0:1, e:e + 1]
        te = te + (j64 >= tb_e).astype(jnp.int32)
    te_ref[...] = te - 1

    sgb = swg_ref[...].astype(jnp.bfloat16)
    sub = swu_ref[...].astype(jnp.bfloat16)
    sdb = swd_ref[...].astype(jnp.bfloat16)
    g = jnp.dot(xb, sgb, preferred_element_type=jnp.float32)
    u = jnp.dot(xb, sub, preferred_element_type=jnp.float32)
    hh = (g * jax.nn.sigmoid(g)) * u
    shared_ref[...] = jnp.dot(hh.astype(jnp.bfloat16), sdb,
                              preferred_element_type=jnp.float32)


def _run_k1(x, gate_w, sw_gate, sw_up, sw_down):
    return pl.pallas_call(
        _k1_body,
        grid=(1,),
        in_specs=[
            pl.BlockSpec((T, H), lambda i: (0, 0)),
            pl.BlockSpec((E, H), lambda i: (0, 0)),
            pl.BlockSpec((H, IS_DIM), lambda i: (0, 0)),
            pl.BlockSpec((H, IS_DIM), lambda i: (0, 0)),
            pl.BlockSpec((IS_DIM, H), lambda i: (0, 0)),
        ],
        out_specs=[
            pl.BlockSpec((T, H), lambda i: (0, 0)),
            pl.BlockSpec((T, H), lambda i: (0, 0)),
            pl.BlockSpec((T, 1), lambda i: (0, 0)),
            pl.BlockSpec((T, 1), lambda i: (0, 0)),
            pl.BlockSpec((T, 1), lambda i: (0, 0)),
            pl.BlockSpec((T, 1), lambda i: (0, 0)),
            pl.BlockSpec((8, 64), lambda i: (0, 0)),
        ],
        out_shape=[
            jax.ShapeDtypeStruct((T, H), jnp.float32),
            jax.ShapeDtypeStruct((T, H), jnp.bfloat16),
            jax.ShapeDtypeStruct((T, 1), jnp.int32),
            jax.ShapeDtypeStruct((T, 1), jnp.int32),
            jax.ShapeDtypeStruct((T, 1), jnp.float32),
            jax.ShapeDtypeStruct((T, 1), jnp.float32),
            jax.ShapeDtypeStruct((8, 64), jnp.int32),
        ],
    )(x, gate_w, sw_gate, sw_up, sw_down)


def _dispatch_xs(xb3d, slots):
    mesh = plsc.VectorSubcoreMesh(core_axis_name="c", subcore_axis_name="s")

    @functools.partial(
        pl.kernel, mesh=mesh,
        out_type=jax.ShapeDtypeStruct((P, 4, 128), jnp.int32),
        scratch_types=[
            pltpu.VMEM((128,), jnp.int32),
            pltpu.VMEM((128, 4, 128), jnp.int32),
            pltpu.SemaphoreType.DMA,
        ],
    )
    def k2(xb_hbm, slots_hbm, xs_hbm, idx_v, rows_v, sem):
        wid = lax.axis_index("s") * 2 + lax.axis_index("c")
        tok0 = (wid % 16) * 128
        pltpu.sync_copy(xb_hbm.at[pl.ds(tok0, 128)], rows_v)
        pltpu.sync_copy(slots_hbm.at[wid], idx_v)
        pltpu.async_copy(rows_v, xs_hbm.at[idx_v], sem).wait()

    return k2(xb3d, slots)


def _k3_body(te_ref, xs_ref, wg_ref, wu_ref, wd_ref, ys_ref):
    xs = xs_ref[...]
    g = jnp.dot(xs, wg_ref[0], preferred_element_type=jnp.float32)
    u = jnp.dot(xs, wu_ref[0], preferred_element_type=jnp.float32)
    hh = (g * jax.nn.sigmoid(g)) * u
    ys_ref[...] = jnp.dot(hh.astype(jnp.bfloat16), wd_ref[0],
                          preferred_element_type=jnp.float32
                          ).astype(jnp.bfloat16)


def _run_k3(te, xs2d, wgb, wub, wdb):
    grid_spec = pltpu.PrefetchScalarGridSpec(
        num_scalar_prefetch=1,
        grid=(NT,),
        in_specs=[
            pl.BlockSpec((TM, H), lambda t, te_r: (t, 0)),
            pl.BlockSpec((1, H, I_DIM), lambda t, te_r: (te_r[t], 0, 0)),
            pl.BlockSpec((1, H, I_DIM), lambda t, te_r: (te_r[t], 0, 0)),
            pl.BlockSpec((1, I_DIM, H), lambda t, te_r: (te_r[t], 0, 0)),
        ],
        out_specs=pl.BlockSpec((TM, H), lambda t, te_r: (t, 0)),
    )
    return pl.pallas_call(
        _k3_body,
        grid_spec=grid_spec,
        out_shape=jax.ShapeDtypeStruct((P, H), jnp.bfloat16),
    )(te, xs2d, wgb, wub, wdb)


def _gather_ab(ys3d, pos_all):
    mesh = plsc.VectorSubcoreMesh(core_axis_name="c", subcore_axis_name="s")

    @functools.partial(
        pl.kernel, mesh=mesh,
        out_type=(jax.ShapeDtypeStruct((T, 4, 128), jnp.int32),
                  jax.ShapeDtypeStruct((T, 4, 128), jnp.int32)),
        scratch_types=[
            pltpu.VMEM((64,), jnp.int32),
            pltpu.VMEM((64, 4, 128), jnp.int32),
            pltpu.SemaphoreType.DMA,
        ],
    )
    def k4a(ys_hbm, pos_hbm, a_hbm, b_hbm, idx_v, rows_v, sem):
        wid = lax.axis_index("s") * 2 + lax.axis_index("c")
        tok0 = wid * 64
        pltpu.sync_copy(pos_hbm.at[wid], idx_v)
        pltpu.async_copy(ys_hbm.at[idx_v], rows_v, sem).wait()
        pltpu.sync_copy(rows_v, a_hbm.at[pl.ds(tok0, 64)])
        pltpu.sync_copy(pos_hbm.at[32 + wid], idx_v)
        pltpu.async_copy(ys_hbm.at[idx_v], rows_v, sem).wait()
        pltpu.sync_copy(rows_v, b_hbm.at[pl.ds(tok0, 64)])

    return k4a(ys3d, pos_all)


def _k4b_body(a_ref, b_ref, sh_ref, w1_ref, w2_ref, out_ref):
    out_ref[...] = (w1_ref[...] * a_ref[...].astype(jnp.float32)
                    + w2_ref[...] * b_ref[...].astype(jnp.float32)
                    + sh_ref[...])


def _run_k4b(a2d, b2d, shared, w1, w2):
    return pl.pallas_call(
        _k4b_body,
        grid=(1,),
        in_specs=[
            pl.BlockSpec((T, H), lambda i: (0, 0)),
            pl.BlockSpec((T, H), lambda i: (0, 0)),
            pl.BlockSpec((T, H), lambda i: (0, 0)),
            pl.BlockSpec((T, 1), lambda i: (0, 0)),
            pl.BlockSpec((T, 1), lambda i: (0, 0)),
        ],
        out_specs=pl.BlockSpec((T, H), lambda i: (0, 0)),
        out_shape=jax.ShapeDtypeStruct((T, H), jnp.float32),
    )(a2d, b2d, shared, w1, w2)


def kernel(hidden_states, gate_w, w_gate, w_up, w_down, sw_gate, sw_up, sw_down):
    b, s, h = hidden_states.shape
    x = hidden_states.reshape(s, h)

    shared, xb, slot0, slot1, w1, w2, te8 = _run_k1(
        x, gate_w, sw_gate, sw_up, sw_down)
    te = te8[0]

    slots = jnp.concatenate([slot0.reshape(16, 128),
                             slot1.reshape(16, 128)], axis=0)
    xb_i = jax.lax.bitcast_convert_type(
        xb.reshape(T, 512, 2), jnp.int32).reshape(T, 4, 128)
    xs3d = _dispatch_xs(xb_i, slots)

    wgb = w_gate.astype(jnp.bfloat16)
    wub = w_up.astype(jnp.bfloat16)
    wdb = w_down.astype(jnp.bfloat16)
    xs2d = jax.lax.bitcast_convert_type(
        xs3d.reshape(P, 512), jnp.bfloat16).reshape(P, H)
    ys = _run_k3(te, xs2d, wgb, wub, wdb)

    pos_all = jnp.concatenate([slot0.reshape(32, 64),
                               slot1.reshape(32, 64)], axis=0)
    ys_i = jax.lax.bitcast_convert_type(
        ys.reshape(P, 512, 2), jnp.int32).reshape(P, 4, 128)
    a3d, b3d = _gather_ab(ys_i, pos_all)
    a2d = jax.lax.bitcast_convert_type(
        a3d.reshape(T, 512), jnp.bfloat16).reshape(T, H)
    b2d = jax.lax.bitcast_convert_type(
        b3d.reshape(T, 512), jnp.bfloat16).reshape(T, H)

    out = _run_k4b(a2d, b2d, shared, w1, w2)
    return out.reshape(b, s, h)


# speedup vs baseline: 3.1822x; 3.1822x over previous
import functools

import jax
import jax.numpy as jnp
from jax import lax
from jax.experimental import pallas as pl
from jax.experimental.pallas import tpu as pltpu
from jax.experimental.pallas import tpu_sc as plsc

E = 8
H = 1024
I_DIM = 512
IS_DIM = 512
T = 2048
TM = 128
NT = 40
P = NT * TM
NSUB = 32


def _k1_body(x_ref, gate_w_ref, swg_ref, swu_ref, swd_ref,
             shared_ref, slot0_ref, slot1_ref, w1_ref, w2_ref,
             te_ref):
    x32 = x_ref[...]
    xb = x32.astype(jnp.bfloat16)

    logits = jax.lax.dot_general(
        x32, gate_w_ref[...], (((1,), (1,)), ((), ())),
        preferred_element_type=jnp.float32)
    m = jnp.max(logits, axis=-1, keepdims=True)
    p = jnp.exp(logits - m)
    p = p / jnp.sum(p, axis=-1, keepdims=True)
    lane = jax.lax.broadcasted_iota(jnp.int32, (T, E), 1)
    v1 = jnp.max(p, axis=-1, keepdims=True)
    e1 = jnp.min(jnp.where(p >= v1, lane, E), axis=-1, keepdims=True)
    m1 = lane == e1
    pm = jnp.where(m1, -jnp.inf, p)
    v2 = jnp.max(pm, axis=-1, keepdims=True)
    e2 = jnp.min(jnp.where(pm >= v2, lane, E), axis=-1, keepdims=True)
    m2 = lane == e2
    s = v1 + v2 + 1e-20
    w1_ref[...] = v1 / s
    w2_ref[...] = v2 / s

    oh1 = m1.astype(jnp.float32)
    oh2 = m2.astype(jnp.float32)

    r = jax.lax.broadcasted_iota(jnp.int32, (TM, TM), 0)
    c = jax.lax.broadcasted_iota(jnp.int32, (TM, TM), 1)
    tril = (r > c).astype(jnp.float32)
    def excl_cumsum(oh):
        out, off = [], jnp.zeros((1, E), jnp.float32)
        for b in range(T // TM):
            blk = oh[b * TM:(b + 1) * TM]
            out.append(jnp.dot(tril, blk, preferred_element_type=jnp.float32)
                       + off)
            off = off + jnp.sum(blk, axis=0, keepdims=True)
        return jnp.concatenate(out, axis=0), off
    C1, tot1 = excl_cumsum(oh1)
    C2, tot2 = excl_cumsum(oh2)

    counts = tot1 + tot2
    ntiles = jnp.floor((counts + (TM - 1)) * (1.0 / TM))
    fe = jax.lax.broadcasted_iota(jnp.int32, (E, E), 0)
    ee = jax.lax.broadcasted_iota(jnp.int32, (E, E), 1)
    lt = (fe < ee).astype(jnp.float32)
    tbase_tiles = jnp.dot(ntiles, lt,
                          preferred_element_type=jnp.float32)
    base = tbase_tiles * float(TM)

    rank0 = jnp.sum(C1 * oh1, axis=-1, keepdims=True)
    rank1 = jnp.sum((tot1 + C2) * oh2, axis=-1, keepdims=True)
    base0 = jnp.sum(base * oh1, axis=-1, keepdims=True)
    base1 = jnp.sum(base * oh2, axis=-1, keepdims=True)
    slot0_ref[...] = (base0 + rank0).astype(jnp.int32)
    slot1_ref[...] = (base1 + rank1).astype(jnp.int32)

    j64 = jax.lax.broadcasted_iota(jnp.int32, (8, 64), 1)
    tbi = tbase_tiles.astype(jnp.int32)
    te = jnp.zeros((8, 64), jnp.int32)
    for e in range(E):
        tb_e = tbi[0:1, e:e + 1]
        te = te + (j64 >= tb_e).astype(jnp.int32)
    te_ref[...] = te - 1

    sgb = swg_ref[...].astype(jnp.bfloat16)
    sub = swu_ref[...].astype(jnp.bfloat16)
    sdb = swd_ref[...].astype(jnp.bfloat16)
    g = jnp.dot(xb, sgb, preferred_element_type=jnp.float32)
    u = jnp.dot(xb, sub, preferred_element_type=jnp.float32)
    hh = (g * jax.nn.sigmoid(g)) * u
    shared_ref[...] = jnp.dot(hh.astype(jnp.bfloat16), sdb,
                              preferred_element_type=jnp.float32)


def _run_k1(x, gate_w, sw_gate, sw_up, sw_down):
    return pl.pallas_call(
        _k1_body,
        grid=(1,),
        in_specs=[
            pl.BlockSpec((T, H), lambda i: (0, 0)),
            pl.BlockSpec((E, H), lambda i: (0, 0)),
            pl.BlockSpec((H, IS_DIM), lambda i: (0, 0)),
            pl.BlockSpec((H, IS_DIM), lambda i: (0, 0)),
            pl.BlockSpec((IS_DIM, H), lambda i: (0, 0)),
        ],
        out_specs=[
            pl.BlockSpec((T, H), lambda i: (0, 0)),
            pl.BlockSpec((T, 1), lambda i: (0, 0)),
            pl.BlockSpec((T, 1), lambda i: (0, 0)),
            pl.BlockSpec((T, 1), lambda i: (0, 0)),
            pl.BlockSpec((T, 1), lambda i: (0, 0)),
            pl.BlockSpec((8, 64), lambda i: (0, 0)),
        ],
        out_shape=[
            jax.ShapeDtypeStruct((T, H), jnp.float32),
            jax.ShapeDtypeStruct((T, 1), jnp.int32),
            jax.ShapeDtypeStruct((T, 1), jnp.int32),
            jax.ShapeDtypeStruct((T, 1), jnp.float32),
            jax.ShapeDtypeStruct((T, 1), jnp.float32),
            jax.ShapeDtypeStruct((8, 64), jnp.int32),
        ],
    )(x, gate_w, sw_gate, sw_up, sw_down)


def _dispatch_xs(x2d, slots):
    mesh = plsc.VectorSubcoreMesh(core_axis_name="c", subcore_axis_name="s")

    @functools.partial(
        pl.kernel, mesh=mesh,
        out_type=jax.ShapeDtypeStruct((P, H), jnp.float32),
        scratch_types=[
            pltpu.VMEM((64,), jnp.int32),
            pltpu.VMEM((64, H), jnp.float32),
            pltpu.SemaphoreType.DMA,
        ],
    )
    def k2(x_hbm, slots_hbm, xs_hbm, idx_v, rows_v, sem):
        wid = lax.axis_index("s") * 2 + lax.axis_index("c")
        k = wid // 16
        w1 = wid % 16
        for c in range(2):
            tok0 = 64 * (2 * w1 + c)
            pltpu.sync_copy(x_hbm.at[pl.ds(tok0, 64)], rows_v)
            pltpu.sync_copy(slots_hbm.at[k * 32 + 2 * w1 + c], idx_v)
            pltpu.async_copy(rows_v, xs_hbm.at[idx_v], sem).wait()

    return k2(x2d, slots)


def _k3_body(te_ref, xs_ref, wg_ref, wu_ref, wd_ref, ys_ref):
    xs = xs_ref[...].astype(jnp.bfloat16)
    g = jnp.dot(xs, wg_ref[0], preferred_element_type=jnp.float32)
    u = jnp.dot(xs, wu_ref[0], preferred_element_type=jnp.float32)
    hh = (g * jax.nn.sigmoid(g)) * u
    ys_ref[...] = jnp.dot(hh.astype(jnp.bfloat16), wd_ref[0],
                          preferred_element_type=jnp.float32)


def _run_k3(te, xs2d, wgb, wub, wdb):
    grid_spec = pltpu.PrefetchScalarGridSpec(
        num_scalar_prefetch=1,
        grid=(NT,),
        in_specs=[
            pl.BlockSpec((TM, H), lambda t, te_r: (t, 0)),
            pl.BlockSpec((1, H, I_DIM), lambda t, te_r: (te_r[t], 0, 0)),
            pl.BlockSpec((1, H, I_DIM), lambda t, te_r: (te_r[t], 0, 0)),
            pl.BlockSpec((1, I_DIM, H), lambda t, te_r: (te_r[t], 0, 0)),
        ],
        out_specs=pl.BlockSpec((TM, H), lambda t, te_r: (t, 0)),
    )
    return pl.pallas_call(
        _k3_body,
        grid_spec=grid_spec,
        out_shape=jax.ShapeDtypeStruct((P, H), jnp.float32),
    )(te, xs2d, wgb, wub, wdb)


def _gather_ab(ys2d, pos_all):
    mesh = plsc.VectorSubcoreMesh(core_axis_name="c", subcore_axis_name="s")

    @functools.partial(
        pl.kernel, mesh=mesh,
        out_type=(jax.ShapeDtypeStruct((T, H), jnp.float32),
                  jax.ShapeDtypeStruct((T, H), jnp.float32)),
        scratch_types=[
            pltpu.VMEM((64,), jnp.int32),
            pltpu.VMEM((64, H), jnp.float32),
            pltpu.SemaphoreType.DMA,
        ],
    )
    def k4a(ys_hbm, pos_hbm, a_hbm, b_hbm, idx_v, rows_v, sem):
        wid = lax.axis_index("s") * 2 + lax.axis_index("c")
        tok0 = wid * 64
        pltpu.sync_copy(pos_hbm.at[wid], idx_v)
        pltpu.async_copy(ys_hbm.at[idx_v], rows_v, sem).wait()
        pltpu.sync_copy(rows_v, a_hbm.at[pl.ds(tok0, 64)])
        pltpu.sync_copy(pos_hbm.at[32 + wid], idx_v)
        pltpu.async_copy(ys_hbm.at[idx_v], rows_v, sem).wait()
        pltpu.sync_copy(rows_v, b_hbm.at[pl.ds(tok0, 64)])

    return k4a(ys2d, pos_all)


def _k4b_body(a_ref, b_ref, sh_ref, w1_ref, w2_ref, out_ref):
    out_ref[...] = (w1_ref[...] * a_ref[...] + w2_ref[...] * b_ref[...]
                    + sh_ref[...])


def _run_k4b(a2d, b2d, shared, w1, w2):
    return pl.pallas_call(
        _k4b_body,
        grid=(1,),
        in_specs=[
            pl.BlockSpec((T, H), lambda i: (0, 0)),
            pl.BlockSpec((T, H), lambda i: (0, 0)),
            pl.BlockSpec((T, H), lambda i: (0, 0)),
            pl.BlockSpec((T, 1), lambda i: (0, 0)),
            pl.BlockSpec((T, 1), lambda i: (0, 0)),
        ],
        out_specs=pl.BlockSpec((T, H), lambda i: (0, 0)),
        out_shape=jax.ShapeDtypeStruct((T, H), jnp.float32),
    )(a2d, b2d, shared, w1, w2)


def kernel(hidden_states, gate_w, w_gate, w_up, w_down, sw_gate, sw_up, sw_down):
    b, s, h = hidden_states.shape
    x = hidden_states.reshape(s, h)

    shared, slot0, slot1, w1, w2, te8 = _run_k1(
        x, gate_w, sw_gate, sw_up, sw_down)
    te = te8[0]

    slots = jnp.concatenate([slot0.reshape(32, 64),
                             slot1.reshape(32, 64)], axis=0)
    xs = _dispatch_xs(x, slots)

    wgb = w_gate.astype(jnp.bfloat16)
    wub = w_up.astype(jnp.bfloat16)
    wdb = w_down.astype(jnp.bfloat16)
    ys = _run_k3(te, xs, wgb, wub, wdb)

    pos_all = jnp.concatenate([slot0.reshape(32, 64),
                               slot1.reshape(32, 64)], axis=0)
    a2d, b2d = _gather_ab(ys, pos_all)

    out = _run_k4b(a2d, b2d, shared, w1, w2)
    return out.reshape(b, s, h)


# speedup vs baseline: 6.1645x; 1.9372x over previous
import jax
import jax.numpy as jnp
from jax.experimental import pallas as pl
from jax.experimental.pallas import tpu as pltpu

E = 8
H = 1024
I_DIM = 512
IS_DIM = 512
T = 2048

_HI = jax.lax.Precision.HIGHEST


def _silu_mul(g, u):
    return (g * jax.nn.sigmoid(g)) * u


def _moe_body(x_ref, gate_w_ref, wg_ref, wu_ref, wd_ref,
              swg_ref, swu_ref, swd_ref, out_ref, comb_ref, xb_ref):
    e = pl.program_id(0)

    @pl.when(e == 0)
    def _router():
        x32 = x_ref[...]
        logits = jax.lax.dot_general(
            x32, gate_w_ref[...], (((1,), (1,)), ((), ())),
            preferred_element_type=jnp.float32)
        m = jnp.max(logits, axis=-1, keepdims=True)
        p = jnp.exp(logits - m)
        p = p / jnp.sum(p, axis=-1, keepdims=True)
        v1 = jnp.max(p, axis=-1, keepdims=True)
        p2 = jnp.where(p >= v1, -jnp.inf, p)
        v2 = jnp.max(p2, axis=-1, keepdims=True)
        s = v1 + v2 + 1e-20
        comb_ref[...] = jnp.where(p >= v1, v1 / s,
                                  jnp.where(p >= v2, v2 / s, 0.0))
        out_ref[...] = jnp.zeros((T, H), jnp.float32)
        xb_ref[...] = x32.astype(jnp.bfloat16)

    xb = xb_ref[...]

    @pl.when(e < E)
    def _routed():
        wgb = wg_ref[0].astype(jnp.bfloat16)
        wub = wu_ref[0].astype(jnp.bfloat16)
        wdb = wd_ref[0].astype(jnp.bfloat16)
        g = jnp.dot(xb, wgb, preferred_element_type=jnp.float32)
        u = jnp.dot(xb, wub, preferred_element_type=jnp.float32)
        lane = jax.lax.broadcasted_iota(jnp.int32, (T, E), 1)
        col = jnp.sum(jnp.where(lane == e, comb_ref[...], 0.0),
                      axis=-1, keepdims=True)
        h = _silu_mul(g, u) * col
        out_ref[...] += jnp.dot(h.astype(jnp.bfloat16), wdb,
                                 preferred_element_type=jnp.float32)

    @pl.when(e == E)
    def _shared():
        sgb = swg_ref[...].astype(jnp.bfloat16)
        sub = swu_ref[...].astype(jnp.bfloat16)
        sdb = swd_ref[...].astype(jnp.bfloat16)
        g = jnp.dot(xb, sgb, preferred_element_type=jnp.float32)
        u = jnp.dot(xb, sub, preferred_element_type=jnp.float32)
        h = _silu_mul(g, u)
        out_ref[...] += jnp.dot(h.astype(jnp.bfloat16), sdb,
                                 preferred_element_type=jnp.float32)


def kernel(hidden_states, gate_w, w_gate, w_up, w_down, sw_gate, sw_up, sw_down):
    b, s, h = hidden_states.shape
    x = hidden_states.reshape(s, h)

    out = pl.pallas_call(
        _moe_body,
        grid=(E + 1,),
        in_specs=[
            pl.BlockSpec((T, H), lambda e: (0, 0)),
            pl.BlockSpec((E, H), lambda e: (0, 0)),
            pl.BlockSpec((1, H, I_DIM), lambda e: (jnp.minimum(e, E - 1), 0, 0)),
            pl.BlockSpec((1, H, I_DIM), lambda e: (jnp.minimum(e, E - 1), 0, 0)),
            pl.BlockSpec((1, I_DIM, H), lambda e: (jnp.minimum(e, E - 1), 0, 0)),
            pl.BlockSpec((H, IS_DIM), lambda e: (0, 0)),
            pl.BlockSpec((H, IS_DIM), lambda e: (0, 0)),
            pl.BlockSpec((IS_DIM, H), lambda e: (0, 0)),
        ],
        out_specs=pl.BlockSpec((T, H), lambda e: (0, 0)),
        out_shape=jax.ShapeDtypeStruct((s, h), jnp.float32),
        scratch_shapes=[
            pltpu.VMEM((T, E), jnp.float32),
            pltpu.VMEM((T, H), jnp.bfloat16),
        ],
    )(x, gate_w, w_gate, w_up, w_down, sw_gate, sw_up, sw_down)
    return out.reshape(b, s, h)
